# Initial kernel scaffold; baseline (speedup 1.0000x reference)
import jax
import jax.numpy as jnp
from jax.experimental import pallas as pl


def kernel(op_ids, node_feats, config_feats, edge_index, emb, gnn0_W, gnn0_b, gnn1_W, gnn1_b, gnn2_W, gnn2_b, shared_W1, shared_b1, shared_W2, shared_b2, post_W1, post_b1, post_W2, post_b2, post_W3, post_b3):
    raise NotImplementedError("write your pallas kernel here")


# speedup vs baseline: 1.0290x; 1.0290x over previous
import jax
import jax.numpy as jnp
from jax.experimental import pallas as pl


def _postnet_body(pooled_ref, w1_ref, b1_ref, w2_ref, b2_ref, w3_ref, b3_ref, out_ref):
    act = lambda t: jax.nn.leaky_relu(t, 0.2)
    h = act(pooled_ref[...] @ w1_ref[...] + b1_ref[...])
    h = act(h @ w2_ref[...] + b2_ref[...])
    out_ref[...] = h @ w3_ref[...] + b3_ref[...]


def kernel(op_ids, node_feats, config_feats, edge_index, emb,
           gnn0_W, gnn0_b, gnn1_W, gnn1_b, gnn2_W, gnn2_b,
           shared_W1, shared_b1, shared_W2, shared_b2,
           post_W1, post_b1, post_W2, post_b2, post_W3, post_b3):
    act = lambda t: jax.nn.leaky_relu(t, 0.2)
    n = node_feats.shape[0]
    c = config_feats.shape[0]
    ope = emb.shape[1]
    draw = node_feats.shape[1]

    W12 = gnn1_W @ gnn2_W
    W = gnn0_W @ W12
    b = gnn0_b @ W12 + gnn1_b @ gnn2_W + gnn2_b
    Wa = W[:ope]
    Wb = W[ope:ope + draw]
    Wc = W[ope + draw:]

    emb2 = emb @ Wa
    u = emb2[op_ids] + node_feats @ Wb
    v = config_feats @ Wc + b
    x = u[:, None, :] + v[None, :, :]

    src = edge_index[0]
    dst = edge_index[1]
    deg = jnp.zeros((n,), jnp.float32).at[dst].add(1.0) + 1.0
    inv_deg = 1.0 / deg

    def shared(t):
        return act(t @ shared_W1 + shared_b1) @ shared_W2 + shared_b2

    for _ in range(3):
        y = act(x)
        yn = y * inv_deg[:, None, None]
        p1 = jnp.zeros_like(y).at[src].add(yn[dst])
        p2 = jnp.zeros_like(y).at[dst].add(y[src])
        z = y + 2.0 * yn + p1 + p2 * inv_deg[:, None, None]
        x = x + shared(z)

    pooled = jnp.sum(act(x), axis=0)

    out = pl.pallas_call(
        _postnet_body,
        out_shape=jax.ShapeDtypeStruct((c, 1), jnp.float32),
    )(pooled, post_W1, post_b1[None, :], post_W2, post_b2[None, :],
      post_W3, post_b3[None, :])
    return jnp.squeeze(out, -1)


# speedup vs baseline: 28.4065x; 27.6050x over previous
import jax
import jax.numpy as jnp
from jax import lax
from jax.experimental import pallas as pl
from jax.experimental.pallas import tpu as pltpu, tpu_sc as plsc

_NFLAT = 512
_Q = 128
_NPAD = 10240
_B = 64
_G = 8
_NCH = 5000
_NGRP = _NCH // _G
_TPG = 40
_ZROWS = _NPAD // 16


def _sc_body(y4_ref, yn4_ref, k1_ref, i1_ref, k2_ref, i2_ref,
             s1_ref, s2_ref, kg, ig, rows, zeros, zsh, sem):
    cid = lax.axis_index("c")
    sid = lax.axis_index("s")

    def zz(r, _):
        for j in range(_Q // 16):
            zeros[r, pl.ds(16 * j, 16)] = jnp.zeros((16,), jnp.float32)
        return 0
    lax.fori_loop(0, _ZROWS // 4, zz, 0)

    for p in range(2):
        qq = 2 * cid + p
        for d in range(2):
            ents_k = (k1_ref, k2_ref)[d]
            ents_i = (i1_ref, i2_ref)[d]
            table = (yn4_ref, y4_ref)[d].at[qq]
            for h in range(4):
                pltpu.sync_copy(
                    zeros, zsh.at[pl.ds(sid * _ZROWS + h * (_ZROWS // 4),
                                        _ZROWS // 4)])
            plsc.subcore_barrier()
            def tloop(t, _):
                g = sid * _TPG + t

                @pl.when(g < _NGRP)
                def _grp():
                    pltpu.sync_copy(ents_k.at[pl.ds(g * _G, _G)], kg)
                    pltpu.sync_copy(ents_i.at[pl.ds(g * _G, _G)], ig)
                    for j in range(_G):
                        pltpu.async_copy(table.at[ig.at[j]], rows,
                                         sem).wait()
                        pltpu.sync_copy(rows, zsh.at[kg.at[j]], add=True)
                return 0
            lax.fori_loop(0, _TPG, tloop, 0)
            plsc.subcore_barrier()
            out = (s1_ref, s2_ref)[d].at[qq]
            pltpu.sync_copy(zsh.at[pl.ds(sid * _ZROWS, _ZROWS)],
                            out.at[pl.ds(sid * _ZROWS, _ZROWS)])
            plsc.subcore_barrier()


def _sc_sparse(y4, yn4, k1, i1, k2, i2):
    mesh = plsc.VectorSubcoreMesh(core_axis_name="c", subcore_axis_name="s")
    f = pl.kernel(
        _sc_body,
        out_type=[jax.ShapeDtypeStruct((4, _NPAD, _Q), jnp.float32),
                  jax.ShapeDtypeStruct((4, _NPAD, _Q), jnp.float32)],
        mesh=mesh,
        scratch_types=[
            pltpu.VMEM((_G, _B), jnp.int32),
            pltpu.VMEM((_G, _B), jnp.int32),
            pltpu.VMEM((_B, _Q), jnp.float32),
            pltpu.VMEM((_ZROWS // 4, _Q), jnp.float32),
            pltpu.VMEM_SHARED((_NPAD, _Q), jnp.float32),
            pltpu.SemaphoreType.DMA,
        ],
    )
    return f(y4, yn4, k1, i1, k2, i2)


_BN = 1000


def _quarters(v):
    return jnp.stack([v[:, i * _Q:(i + 1) * _Q] for i in range(4)], axis=0)


def _init_body(op_ref, nf_ref, emb2_ref, wb_ref, vflat_ref, inv_ref,
               x_ref, y_ref, yn_ref):
    op = op_ref[...]
    onehot = (op == lax.broadcasted_iota(jnp.int32, (1, 128), 1)
              ).astype(jnp.float32)
    u = (jnp.dot(onehot, emb2_ref[...], preferred_element_type=jnp.float32)
         + jnp.dot(nf_ref[...], wb_ref[...], preferred_element_type=jnp.float32))
    x = jnp.concatenate([u] * 8, axis=1) + vflat_ref[...]
    x_ref[...] = x
    y = jnp.where(x >= 0, x, 0.2 * x)
    y_ref[...] = _quarters(y)
    yn_ref[...] = _quarters(y * inv_ref[...])


def _tc_init(op_ids2, node_feats, emb2, wb, vflat, inv2):
    n = node_feats.shape[0]
    grid = n // _BN
    outx = jax.ShapeDtypeStruct((n, _NFLAT), jnp.float32)
    out4 = jax.ShapeDtypeStruct((4, n, _Q), jnp.float32)
    blk4 = pl.BlockSpec((4, _BN, _Q), lambda i: (0, i, 0))
    return pl.pallas_call(
        _init_body,
        grid=(grid,),
        in_specs=[
            pl.BlockSpec((_BN, 1), lambda i: (i, 0)),
            pl.BlockSpec((_BN, 128), lambda i: (i, 0)),
            pl.BlockSpec((128, 64), lambda i: (0, 0)),
            pl.BlockSpec((128, 64), lambda i: (0, 0)),
            pl.BlockSpec((1, _NFLAT), lambda i: (0, 0)),
            pl.BlockSpec((_BN, 1), lambda i: (i, 0)),
        ],
        out_specs=[pl.BlockSpec((_BN, _NFLAT), lambda i: (i, 0)), blk4, blk4],
        out_shape=[outx, out4, out4],
    )(op_ids2, node_feats, emb2, wb, vflat, inv2)


def _upd_body(x_ref, s1_ref, s2_ref, y_ref, yn_ref, inv_ref,
              w1_ref, b1_ref, w2_ref, b2_ref, xn_ref, y2_ref, yn2_ref):
    act = lambda t: jnp.where(t >= 0, t, 0.2 * t)
    inv = inv_ref[...]
    cat = lambda a: jnp.concatenate([a[i] for i in range(4)], axis=-1)
    z = (cat(s1_ref[...]) + cat(s2_ref[...]) * inv
         + cat(y_ref[...]) + 2.0 * cat(yn_ref[...]))
    t = jnp.dot(act(jnp.dot(z, w1_ref[...], preferred_element_type=jnp.float32)
                    + b1_ref[...]),
                w2_ref[...], preferred_element_type=jnp.float32) + b2_ref[...]
    xn = x_ref[...] + t
    xn_ref[...] = xn
    y2 = act(xn)
    y2_ref[...] = _quarters(y2)
    yn2_ref[...] = _quarters(y2 * inv)


def _tc_update(x, s1, s2, y4, yn4, inv2, w1bd, b1t, w2bd, b2t):
    n = x.shape[0]
    grid = n // _BN
    outx = jax.ShapeDtypeStruct((n, _NFLAT), jnp.float32)
    out4 = jax.ShapeDtypeStruct((4, n, _Q), jnp.float32)
    blkx = pl.BlockSpec((_BN, _NFLAT), lambda i: (i, 0))
    blk4 = pl.BlockSpec((4, _BN, _Q), lambda i: (0, i, 0))
    return pl.pallas_call(
        _upd_body,
        grid=(grid,),
        in_specs=[
            blkx, blk4, blk4, blk4, blk4,
            pl.BlockSpec((_BN, 1), lambda i: (i, 0)),
            pl.BlockSpec((_NFLAT, _NFLAT), lambda i: (0, 0)),
            pl.BlockSpec((1, _NFLAT), lambda i: (0, 0)),
            pl.BlockSpec((_NFLAT, _NFLAT), lambda i: (0, 0)),
            pl.BlockSpec((1, _NFLAT), lambda i: (0, 0)),
        ],
        out_specs=[blkx, blk4, blk4],
        out_shape=[outx, out4, out4],
    )(x, s1, s2, y4, yn4, inv2, w1bd, b1t, w2bd, b2t)


def _pool_body(y_ref, out_ref):
    i = pl.program_id(0)

    @pl.when(i == 0)
    def _():
        out_ref[...] = jnp.zeros_like(out_ref)
    out_ref[...] += jnp.sum(y_ref[...], axis=1, keepdims=True)


def _tc_pool(y4):
    n = y4.shape[1]
    grid = n // _BN
    return pl.pallas_call(
        _pool_body,
        grid=(grid,),
        in_specs=[pl.BlockSpec((4, _BN, _Q), lambda i: (0, i, 0))],
        out_specs=pl.BlockSpec((4, 1, _Q), lambda i: (0, 0, 0)),
        out_shape=jax.ShapeDtypeStruct((4, 1, _Q), jnp.float32),
    )(y4)


def _post_body(pooled_ref, w1_ref, b1_ref, w2_ref, b2_ref, w3_ref, b3_ref,
               out_ref):
    act = lambda t: jnp.where(t >= 0, t, 0.2 * t)
    h = act(jnp.dot(pooled_ref[...], w1_ref[...],
                    preferred_element_type=jnp.float32) + b1_ref[...])
    h = act(jnp.dot(h, w2_ref[...], preferred_element_type=jnp.float32)
            + b2_ref[...])
    out_ref[...] = jnp.dot(h, w3_ref[...],
                           preferred_element_type=jnp.float32) + b3_ref[...]


def kernel(op_ids, node_feats, config_feats, edge_index, emb,
           gnn0_W, gnn0_b, gnn1_W, gnn1_b, gnn2_W, gnn2_b,
           shared_W1, shared_b1, shared_W2, shared_b2,
           post_W1, post_b1, post_W2, post_b2, post_W3, post_b3):
    n = node_feats.shape[0]
    c = config_feats.shape[0]
    ope = emb.shape[1]
    draw = node_feats.shape[1]

    w12 = gnn1_W @ gnn2_W
    wfold = gnn0_W @ w12
    bfold = gnn0_b @ w12 + gnn1_b @ gnn2_W + gnn2_b
    emb2 = emb @ wfold[:ope]
    wb = wfold[ope:ope + draw]
    vflat = (config_feats @ wfold[ope + draw:] + bfold).reshape(1, _NFLAT)

    src = edge_index[0].astype(jnp.int32)
    dst = edge_index[1].astype(jnp.int32)

    p1 = jnp.sort((src << 14) | dst)
    p2 = jnp.sort((dst << 14) | src)
    k1 = (p1 >> 14).astype(jnp.int32).reshape(_NCH, _B)
    i1 = (p1 & 16383).astype(jnp.int32).reshape(_NCH, _B)
    k2 = (p2 >> 14).astype(jnp.int32).reshape(_NCH, _B)
    i2 = (p2 & 16383).astype(jnp.int32).reshape(_NCH, _B)

    node_edges = jnp.searchsorted(
        k2.reshape(-1), jnp.arange(n + 1, dtype=jnp.int32)).astype(jnp.int32)
    deg = (node_edges[1:] - node_edges[:-1]).astype(jnp.float32) + 1.0
    inv2 = (1.0 / deg).reshape(n, 1)

    eye = jnp.eye(c, dtype=jnp.float32)
    w1bd = jnp.kron(eye, shared_W1)
    w2bd = jnp.kron(eye, shared_W2)
    b1t = jnp.tile(shared_b1, c).reshape(1, _NFLAT)
    b2t = jnp.tile(shared_b2, c).reshape(1, _NFLAT)

    x, y4, yn4 = _tc_init(op_ids.reshape(n, 1).astype(jnp.int32),
                          node_feats, emb2, wb, vflat, inv2)

    for _ in range(3):
        s1, s2 = _sc_sparse(y4, yn4, k1, i1, k2, i2)
        x, y4, yn4 = _tc_update(x, s1[:, :n], s2[:, :n], y4, yn4, inv2,
                                w1bd, b1t, w2bd, b2t)

    pooled = _tc_pool(y4).reshape(c, 64)

    out = pl.pallas_call(
        _post_body,
        out_shape=jax.ShapeDtypeStruct((c, 1), jnp.float32),
    )(pooled, post_W1, post_b1[None, :], post_W2, post_b2[None, :],
      post_W3, post_b3[None, :])
    return jnp.squeeze(out, -1)


# speedup vs baseline: 42.0037x; 1.4787x over previous
import jax
import jax.numpy as jnp
from jax import lax
from jax.experimental import pallas as pl
from jax.experimental.pallas import tpu as pltpu, tpu_sc as plsc

_NFLAT = 512
_Q = 128
_NPAD = 10240
_B = 64
_G = 8
_NCH = 5000
_NGRP = 625
_TPG = 40
_ZROWS = _NPAD // 16


def _sc_body(y4_ref, yn4_ref, k1_ref, i1_ref, k2_ref, i2_ref,
             s1_ref, s2_ref, kg, ig, rows, zeros, zsh, gs0, gs1):
    cid = lax.axis_index("c")
    sid = lax.axis_index("s")
    gsem = (gs0, gs1)

    def zz(r, _):
        for j in range(_Q // 16):
            zeros[r, pl.ds(16 * j, 16)] = jnp.zeros((16,), jnp.float32)
        return 0
    lax.fori_loop(0, 160, zz, 0)

    for p in range(2):
        qq = 2 * cid + p
        for d in range(2):
            ents_k = (k1_ref, k2_ref)[d]
            ents_i = (i1_ref, i2_ref)[d]
            table = (yn4_ref, y4_ref)[d].at[qq]
            out = (s1_ref, s2_ref)[d].at[qq]
            for h in range(4):
                pltpu.sync_copy(
                    zeros, zsh.at[pl.ds(sid * _ZROWS + h * 160, 160)])
            plsc.subcore_barrier()

            def tloop(t, _):
                g = sid * _TPG + t

                @pl.when(g < _NGRP)
                def _grp():
                    pltpu.sync_copy(ents_k.at[pl.ds(g * _G, _G)], kg)
                    pltpu.sync_copy(ents_i.at[pl.ds(g * _G, _G)], ig)
                    descs = [None] * _G
                    descs[0] = pltpu.async_copy(
                        table.at[ig.at[0]], rows.at[0], gsem[0])
                    for j in range(_G):
                        if j + 1 < _G:
                            descs[j + 1] = pltpu.async_copy(
                                table.at[ig.at[j + 1]],
                                rows.at[(j + 1) % 2], gsem[(j + 1) % 2])
                        descs[j].wait()
                        pltpu.sync_copy(rows.at[j % 2], zsh.at[kg.at[j]],
                                        add=True)
                return 0
            lax.fori_loop(0, _TPG, tloop, 0)
            plsc.subcore_barrier()
            pltpu.sync_copy(zsh.at[pl.ds(sid * _ZROWS, _ZROWS)],
                            out.at[pl.ds(sid * _ZROWS, _ZROWS)])
            plsc.subcore_barrier()


def _sc_sparse(y4, yn4, k1, i1, k2, i2):
    mesh = plsc.VectorSubcoreMesh(core_axis_name="c", subcore_axis_name="s")
    f = pl.kernel(
        _sc_body,
        out_type=[jax.ShapeDtypeStruct((4, _NPAD, _Q), jnp.float32),
                  jax.ShapeDtypeStruct((4, _NPAD, _Q), jnp.float32)],
        mesh=mesh,
        scratch_types=[
            pltpu.VMEM((_G, _B), jnp.int32),
            pltpu.VMEM((_G, _B), jnp.int32),
            pltpu.VMEM((2, _B, _Q), jnp.float32),
            pltpu.VMEM((160, _Q), jnp.float32),
            pltpu.VMEM_SHARED((_NPAD, _Q), jnp.float32),
            pltpu.SemaphoreType.DMA,
            pltpu.SemaphoreType.DMA,
        ],
    )
    return f(y4, yn4, k1, i1, k2, i2)


_BN = 1000


def _quarters(v):
    return jnp.stack([v[:, i * _Q:(i + 1) * _Q] for i in range(4)], axis=0)


def _init_body(op_ref, nf_ref, emb2_ref, wb_ref, vflat_ref, inv_ref,
               x_ref, y_ref, yn_ref):
    op = op_ref[...]
    onehot = (op == lax.broadcasted_iota(jnp.int32, (1, 128), 1)
              ).astype(jnp.float32)
    u = (jnp.dot(onehot, emb2_ref[...], preferred_element_type=jnp.float32)
         + jnp.dot(nf_ref[...], wb_ref[...], preferred_element_type=jnp.float32))
    x = jnp.concatenate([u] * 8, axis=1) + vflat_ref[...]
    x_ref[...] = x
    y = jnp.where(x >= 0, x, 0.2 * x)
    y_ref[...] = _quarters(y)
    yn_ref[...] = _quarters(y * inv_ref[...])


def _tc_init(op_ids2, node_feats, emb2, wb, vflat, inv2):
    n = node_feats.shape[0]
    grid = n // _BN
    outx = jax.ShapeDtypeStruct((n, _NFLAT), jnp.float32)
    out4 = jax.ShapeDtypeStruct((4, n, _Q), jnp.float32)
    blk4 = pl.BlockSpec((4, _BN, _Q), lambda i: (0, i, 0))
    return pl.pallas_call(
        _init_body,
        grid=(grid,),
        in_specs=[
            pl.BlockSpec((_BN, 1), lambda i: (i, 0)),
            pl.BlockSpec((_BN, 128), lambda i: (i, 0)),
            pl.BlockSpec((128, 64), lambda i: (0, 0)),
            pl.BlockSpec((128, 64), lambda i: (0, 0)),
            pl.BlockSpec((1, _NFLAT), lambda i: (0, 0)),
            pl.BlockSpec((_BN, 1), lambda i: (i, 0)),
        ],
        out_specs=[pl.BlockSpec((_BN, _NFLAT), lambda i: (i, 0)), blk4, blk4],
        out_shape=[outx, out4, out4],
    )(op_ids2, node_feats, emb2, wb, vflat, inv2)


def _upd_body(x_ref, s1_ref, s2_ref, y_ref, yn_ref, inv_ref,
              w1_ref, b1_ref, w2_ref, b2_ref, xn_ref, y2_ref, yn2_ref):
    act = lambda t: jnp.where(t >= 0, t, 0.2 * t)
    inv = inv_ref[...]
    cat = lambda a: jnp.concatenate([a[i] for i in range(4)], axis=-1)
    z = (cat(s1_ref[...]) + cat(s2_ref[...]) * inv
         + cat(y_ref[...]) + 2.0 * cat(yn_ref[...]))
    t = jnp.dot(act(jnp.dot(z, w1_ref[...], preferred_element_type=jnp.float32)
                    + b1_ref[...]),
                w2_ref[...], preferred_element_type=jnp.float32) + b2_ref[...]
    xn = x_ref[...] + t
    xn_ref[...] = xn
    y2 = act(xn)
    y2_ref[...] = _quarters(y2)
    yn2_ref[...] = _quarters(y2 * inv)


def _tc_update(x, s1, s2, y4, yn4, inv2, w1bd, b1t, w2bd, b2t):
    n = x.shape[0]
    grid = n // _BN
    outx = jax.ShapeDtypeStruct((n, _NFLAT), jnp.float32)
    out4 = jax.ShapeDtypeStruct((4, n, _Q), jnp.float32)
    blkx = pl.BlockSpec((_BN, _NFLAT), lambda i: (i, 0))
    blk4 = pl.BlockSpec((4, _BN, _Q), lambda i: (0, i, 0))
    return pl.pallas_call(
        _upd_body,
        grid=(grid,),
        in_specs=[
            blkx, blk4, blk4, blk4, blk4,
            pl.BlockSpec((_BN, 1), lambda i: (i, 0)),
            pl.BlockSpec((_NFLAT, _NFLAT), lambda i: (0, 0)),
            pl.BlockSpec((1, _NFLAT), lambda i: (0, 0)),
            pl.BlockSpec((_NFLAT, _NFLAT), lambda i: (0, 0)),
            pl.BlockSpec((1, _NFLAT), lambda i: (0, 0)),
        ],
        out_specs=[blkx, blk4, blk4],
        out_shape=[outx, out4, out4],
    )(x, s1, s2, y4, yn4, inv2, w1bd, b1t, w2bd, b2t)


def _pool_body(y_ref, out_ref):
    i = pl.program_id(0)

    @pl.when(i == 0)
    def _():
        out_ref[...] = jnp.zeros_like(out_ref)
    out_ref[...] += jnp.sum(y_ref[...], axis=1, keepdims=True)


def _tc_pool(y4):
    n = y4.shape[1]
    grid = n // _BN
    return pl.pallas_call(
        _pool_body,
        grid=(grid,),
        in_specs=[pl.BlockSpec((4, _BN, _Q), lambda i: (0, i, 0))],
        out_specs=pl.BlockSpec((4, 1, _Q), lambda i: (0, 0, 0)),
        out_shape=jax.ShapeDtypeStruct((4, 1, _Q), jnp.float32),
    )(y4)


def _post_body(pooled_ref, w1_ref, b1_ref, w2_ref, b2_ref, w3_ref, b3_ref,
               out_ref):
    act = lambda t: jnp.where(t >= 0, t, 0.2 * t)
    h = act(jnp.dot(pooled_ref[...], w1_ref[...],
                    preferred_element_type=jnp.float32) + b1_ref[...])
    h = act(jnp.dot(h, w2_ref[...], preferred_element_type=jnp.float32)
            + b2_ref[...])
    out_ref[...] = jnp.dot(h, w3_ref[...],
                           preferred_element_type=jnp.float32) + b3_ref[...]


def kernel(op_ids, node_feats, config_feats, edge_index, emb,
           gnn0_W, gnn0_b, gnn1_W, gnn1_b, gnn2_W, gnn2_b,
           shared_W1, shared_b1, shared_W2, shared_b2,
           post_W1, post_b1, post_W2, post_b2, post_W3, post_b3):
    n = node_feats.shape[0]
    c = config_feats.shape[0]
    ope = emb.shape[1]
    draw = node_feats.shape[1]

    w12 = gnn1_W @ gnn2_W
    wfold = gnn0_W @ w12
    bfold = gnn0_b @ w12 + gnn1_b @ gnn2_W + gnn2_b
    emb2 = emb @ wfold[:ope]
    wb = wfold[ope:ope + draw]
    vflat = (config_feats @ wfold[ope + draw:] + bfold).reshape(1, _NFLAT)

    src = edge_index[0].astype(jnp.int32)
    dst = edge_index[1].astype(jnp.int32)

    p2 = jnp.sort((dst << 14) | src)
    k2s = (p2 >> 14).astype(jnp.int32)
    i2s = (p2 & 16383).astype(jnp.int32)

    node_edges = jnp.searchsorted(
        k2s, jnp.arange(n + 1, dtype=jnp.int32)).astype(jnp.int32)
    deg = (node_edges[1:] - node_edges[:-1]).astype(jnp.float32) + 1.0
    inv2 = (1.0 / deg).reshape(n, 1)

    k1 = src.reshape(_NCH, _B)
    i1 = dst.reshape(_NCH, _B)
    k2 = k2s.reshape(_NCH, _B)
    i2 = i2s.reshape(_NCH, _B)

    eye = jnp.eye(c, dtype=jnp.float32)
    w1bd = jnp.kron(eye, shared_W1)
    w2bd = jnp.kron(eye, shared_W2)
    b1t = jnp.tile(shared_b1, c).reshape(1, _NFLAT)
    b2t = jnp.tile(shared_b2, c).reshape(1, _NFLAT)

    x, y4, yn4 = _tc_init(op_ids.reshape(n, 1).astype(jnp.int32),
                          node_feats, emb2, wb, vflat, inv2)

    for _ in range(3):
        s1, s2 = _sc_sparse(y4, yn4, k1, i1, k2, i2)
        x, y4, yn4 = _tc_update(x, s1, s2, y4, yn4, inv2,
                                w1bd, b1t, w2bd, b2t)

    pooled = _tc_pool(y4).reshape(c, 64)

    out = pl.pallas_call(
        _post_body,
        out_shape=jax.ShapeDtypeStruct((c, 1), jnp.float32),
    )(pooled, post_W1, post_b1[None, :], post_W2, post_b2[None, :],
      post_W3, post_b3[None, :])
    return jnp.squeeze(out, -1)


# speedup vs baseline: 43.5400x; 1.0366x over previous
import jax
import jax.numpy as jnp
from jax import lax
from jax.experimental import pallas as pl
from jax.experimental.pallas import tpu as pltpu, tpu_sc as plsc

_NFLAT = 512
_Q = 128
_NPAD = 10240
_B = 64
_G = 8
_NCH = 5000
_NGRP = 625
_TPG = 40
_ZROWS = _NPAD // 16


def _sc_body(y4_ref, yn4_ref, k1_ref, i1_ref, k2_ref, i2_ref, zz_ref,
             s1_ref, s2_ref, kg, ig, rows, zsh,
             gs0, gs1, gs2, gs3, ss0, ss1, ss2, ss3):
    cid = lax.axis_index("c")
    sid = lax.axis_index("s")
    gsem = (gs0, gs1, gs2, gs3)
    ssem = (ss0, ss1, ss2, ss3)

    for p in range(2):
        qq = 2 * cid + p
        for d in range(2):
            ents_k = (k1_ref, k2_ref)[d]
            ents_i = (i1_ref, i2_ref)[d]
            table = (yn4_ref, y4_ref)[d].at[qq]
            out = (s1_ref, s2_ref)[d].at[qq]
            pltpu.sync_copy(zz_ref, zsh.at[pl.ds(sid * _ZROWS, _ZROWS)])
            plsc.subcore_barrier()

            def tloop(t, _):
                g = sid * _TPG + t

                @pl.when(g < _NGRP)
                def _grp():
                    pltpu.sync_copy(ents_k.at[pl.ds(g * _G, _G)], kg)
                    pltpu.sync_copy(ents_i.at[pl.ds(g * _G, _G)], ig)
                    gd = [None] * _G
                    sd = [None] * _G
                    gd[0] = pltpu.async_copy(
                        table.at[ig.at[0]], rows.at[0], gsem[0])
                    for j in range(_G):
                        if j + 1 < _G:
                            if j + 1 >= 4:
                                sd[j - 3].wait()
                            gd[j + 1] = pltpu.async_copy(
                                table.at[ig.at[j + 1]],
                                rows.at[(j + 1) % 4], gsem[(j + 1) % 4])
                        gd[j].wait()
                        sd[j] = pltpu.async_copy(
                            rows.at[j % 4], zsh.at[kg.at[j]],
                            ssem[j % 4], add=True)
                    for j in range(_G - 4, _G):
                        sd[j].wait()
                return 0
            lax.fori_loop(0, _TPG, tloop, 0)
            plsc.subcore_barrier()
            pltpu.sync_copy(zsh.at[pl.ds(sid * _ZROWS, _ZROWS)],
                            out.at[pl.ds(sid * _ZROWS, _ZROWS)])
            plsc.subcore_barrier()


def _sc_sparse(y4, yn4, k1, i1, k2, i2, zz):
    mesh = plsc.VectorSubcoreMesh(core_axis_name="c", subcore_axis_name="s")
    f = pl.kernel(
        _sc_body,
        out_type=[jax.ShapeDtypeStruct((4, _NPAD, _Q), jnp.float32),
                  jax.ShapeDtypeStruct((4, _NPAD, _Q), jnp.float32)],
        mesh=mesh,
        scratch_types=[
            pltpu.VMEM((_G, _B), jnp.int32),
            pltpu.VMEM((_G, _B), jnp.int32),
            pltpu.VMEM((4, _B, _Q), jnp.float32),
            pltpu.VMEM_SHARED((_NPAD, _Q), jnp.float32),
            pltpu.SemaphoreType.DMA,
            pltpu.SemaphoreType.DMA,
            pltpu.SemaphoreType.DMA,
            pltpu.SemaphoreType.DMA,
            pltpu.SemaphoreType.DMA,
            pltpu.SemaphoreType.DMA,
            pltpu.SemaphoreType.DMA,
            pltpu.SemaphoreType.DMA,
        ],
    )
    return f(y4, yn4, k1, i1, k2, i2, zz)


_BN = 1000


def _quarters(v):
    return jnp.stack([v[:, i * _Q:(i + 1) * _Q] for i in range(4)], axis=0)


def _init_body(op_ref, nf_ref, emb2_ref, wb_ref, vflat_ref, inv_ref,
               x_ref, y_ref, yn_ref):
    op = op_ref[...]
    onehot = (op == lax.broadcasted_iota(jnp.int32, (1, 128), 1)
              ).astype(jnp.float32)
    u = (jnp.dot(onehot, emb2_ref[...], preferred_element_type=jnp.float32)
         + jnp.dot(nf_ref[...], wb_ref[...], preferred_element_type=jnp.float32))
    x = jnp.concatenate([u] * 8, axis=1) + vflat_ref[...]
    x_ref[...] = x
    y = jnp.where(x >= 0, x, 0.2 * x)
    y_ref[...] = _quarters(y)
    yn_ref[...] = _quarters(y * inv_ref[...])


def _tc_init(op_ids2, node_feats, emb2, wb, vflat, inv2):
    n = node_feats.shape[0]
    grid = n // _BN
    outx = jax.ShapeDtypeStruct((n, _NFLAT), jnp.float32)
    out4 = jax.ShapeDtypeStruct((4, n, _Q), jnp.float32)
    blk4 = pl.BlockSpec((4, _BN, _Q), lambda i: (0, i, 0))
    return pl.pallas_call(
        _init_body,
        grid=(grid,),
        in_specs=[
            pl.BlockSpec((_BN, 1), lambda i: (i, 0)),
            pl.BlockSpec((_BN, 128), lambda i: (i, 0)),
            pl.BlockSpec((128, 64), lambda i: (0, 0)),
            pl.BlockSpec((128, 64), lambda i: (0, 0)),
            pl.BlockSpec((1, _NFLAT), lambda i: (0, 0)),
            pl.BlockSpec((_BN, 1), lambda i: (i, 0)),
        ],
        out_specs=[pl.BlockSpec((_BN, _NFLAT), lambda i: (i, 0)), blk4, blk4],
        out_shape=[outx, out4, out4],
    )(op_ids2, node_feats, emb2, wb, vflat, inv2)


def _upd_body(x_ref, s1_ref, s2_ref, y_ref, yn_ref, inv_ref,
              w1_ref, b1_ref, w2_ref, b2_ref, xn_ref, y2_ref, yn2_ref):
    act = lambda t: jnp.where(t >= 0, t, 0.2 * t)
    inv = inv_ref[...]
    cat = lambda a: jnp.concatenate([a[i] for i in range(4)], axis=-1)
    z = (cat(s1_ref[...]) + cat(s2_ref[...]) * inv
         + cat(y_ref[...]) + 2.0 * cat(yn_ref[...]))
    t = jnp.dot(act(jnp.dot(z, w1_ref[...], preferred_element_type=jnp.float32)
                    + b1_ref[...]),
                w2_ref[...], preferred_element_type=jnp.float32) + b2_ref[...]
    xn = x_ref[...] + t
    xn_ref[...] = xn
    y2 = act(xn)
    y2_ref[...] = _quarters(y2)
    yn2_ref[...] = _quarters(y2 * inv)


def _tc_update(x, s1, s2, y4, yn4, inv2, w1bd, b1t, w2bd, b2t):
    n = x.shape[0]
    grid = n // _BN
    outx = jax.ShapeDtypeStruct((n, _NFLAT), jnp.float32)
    out4 = jax.ShapeDtypeStruct((4, n, _Q), jnp.float32)
    blkx = pl.BlockSpec((_BN, _NFLAT), lambda i: (i, 0))
    blk4 = pl.BlockSpec((4, _BN, _Q), lambda i: (0, i, 0))
    return pl.pallas_call(
        _upd_body,
        grid=(grid,),
        in_specs=[
            blkx, blk4, blk4, blk4, blk4,
            pl.BlockSpec((_BN, 1), lambda i: (i, 0)),
            pl.BlockSpec((_NFLAT, _NFLAT), lambda i: (0, 0)),
            pl.BlockSpec((1, _NFLAT), lambda i: (0, 0)),
            pl.BlockSpec((_NFLAT, _NFLAT), lambda i: (0, 0)),
            pl.BlockSpec((1, _NFLAT), lambda i: (0, 0)),
        ],
        out_specs=[blkx, blk4, blk4],
        out_shape=[outx, out4, out4],
    )(x, s1, s2, y4, yn4, inv2, w1bd, b1t, w2bd, b2t)


def _pool_body(y_ref, out_ref):
    i = pl.program_id(0)

    @pl.when(i == 0)
    def _():
        out_ref[...] = jnp.zeros_like(out_ref)
    out_ref[...] += jnp.sum(y_ref[...], axis=1, keepdims=True)


def _tc_pool(y4):
    n = y4.shape[1]
    grid = n // _BN
    return pl.pallas_call(
        _pool_body,
        grid=(grid,),
        in_specs=[pl.BlockSpec((4, _BN, _Q), lambda i: (0, i, 0))],
        out_specs=pl.BlockSpec((4, 1, _Q), lambda i: (0, 0, 0)),
        out_shape=jax.ShapeDtypeStruct((4, 1, _Q), jnp.float32),
    )(y4)


def _post_body(pooled_ref, w1_ref, b1_ref, w2_ref, b2_ref, w3_ref, b3_ref,
               out_ref):
    act = lambda t: jnp.where(t >= 0, t, 0.2 * t)
    h = act(jnp.dot(pooled_ref[...], w1_ref[...],
                    preferred_element_type=jnp.float32) + b1_ref[...])
    h = act(jnp.dot(h, w2_ref[...], preferred_element_type=jnp.float32)
            + b2_ref[...])
    out_ref[...] = jnp.dot(h, w3_ref[...],
                           preferred_element_type=jnp.float32) + b3_ref[...]


def kernel(op_ids, node_feats, config_feats, edge_index, emb,
           gnn0_W, gnn0_b, gnn1_W, gnn1_b, gnn2_W, gnn2_b,
           shared_W1, shared_b1, shared_W2, shared_b2,
           post_W1, post_b1, post_W2, post_b2, post_W3, post_b3):
    n = node_feats.shape[0]
    c = config_feats.shape[0]
    ope = emb.shape[1]
    draw = node_feats.shape[1]

    w12 = gnn1_W @ gnn2_W
    wfold = gnn0_W @ w12
    bfold = gnn0_b @ w12 + gnn1_b @ gnn2_W + gnn2_b
    emb2 = emb @ wfold[:ope]
    wb = wfold[ope:ope + draw]
    vflat = (config_feats @ wfold[ope + draw:] + bfold).reshape(1, _NFLAT)

    src = edge_index[0].astype(jnp.int32)
    dst = edge_index[1].astype(jnp.int32)

    p2 = jnp.sort((dst << 14) | src)
    k2s = (p2 >> 14).astype(jnp.int32)
    i2s = (p2 & 16383).astype(jnp.int32)

    node_edges = jnp.searchsorted(
        k2s, jnp.arange(n + 1, dtype=jnp.int32)).astype(jnp.int32)
    deg = (node_edges[1:] - node_edges[:-1]).astype(jnp.float32) + 1.0
    inv2 = (1.0 / deg).reshape(n, 1)

    k1 = src.reshape(_NCH, _B)
    i1 = dst.reshape(_NCH, _B)
    k2 = k2s.reshape(_NCH, _B)
    i2 = i2s.reshape(_NCH, _B)
    zz = jnp.zeros((_ZROWS, _Q), jnp.float32)

    eye = jnp.eye(c, dtype=jnp.float32)
    w1bd = jnp.kron(eye, shared_W1)
    w2bd = jnp.kron(eye, shared_W2)
    b1t = jnp.tile(shared_b1, c).reshape(1, _NFLAT)
    b2t = jnp.tile(shared_b2, c).reshape(1, _NFLAT)

    x, y4, yn4 = _tc_init(op_ids.reshape(n, 1).astype(jnp.int32),
                          node_feats, emb2, wb, vflat, inv2)

    for _ in range(3):
        s1, s2 = _sc_sparse(y4, yn4, k1, i1, k2, i2, zz)
        x, y4, yn4 = _tc_update(x, s1, s2, y4, yn4, inv2,
                                w1bd, b1t, w2bd, b2t)

    pooled = _tc_pool(y4).reshape(c, 64)

    out = pl.pallas_call(
        _post_body,
        out_shape=jax.ShapeDtypeStruct((c, 1), jnp.float32),
    )(pooled, post_W1, post_b1[None, :], post_W2, post_b2[None, :],
      post_W3, post_b3[None, :])
    return jnp.squeeze(out, -1)


# speedup vs baseline: 45.4080x; 1.0429x over previous
import jax
import jax.numpy as jnp
from jax import lax
from jax.experimental import pallas as pl
from jax.experimental.pallas import tpu as pltpu, tpu_sc as plsc

_NFLAT = 512
_Q = 128
_NPAD = 10240
_B = 64
_G = 8
_NCH = 5000
_NGRP = 625
_TPG = 40
_ZROWS = _NPAD // 16


def _sc_body(y4_ref, yn4_ref, ki1_ref, ki2_ref, zz_ref,
             s1_ref, s2_ref, kig, rows, zsh,
             gs0, gs1, gs2, gs3, ss0, ss1, ss2, ss3):
    cid = lax.axis_index("c")
    sid = lax.axis_index("s")
    gsem = (gs0, gs1, gs2, gs3)
    ssem = (ss0, ss1, ss2, ss3)

    for p in range(2):
        qq = 2 * cid + p
        for d in range(2):
            ents = (ki1_ref, ki2_ref)[d]
            table = (yn4_ref, y4_ref)[d].at[qq]
            out = (s1_ref, s2_ref)[d].at[qq]
            pltpu.sync_copy(zz_ref, zsh.at[pl.ds(sid * _ZROWS, _ZROWS)])
            plsc.subcore_barrier()

            def tloop(t, _):
                g = sid * _TPG + t

                @pl.when(g < _NGRP)
                def _grp():
                    pltpu.sync_copy(ents.at[pl.ds(2 * g * _G, 2 * _G)], kig)
                    gd = [None] * _G
                    sd = [None] * _G
                    gd[0] = pltpu.async_copy(
                        table.at[kig.at[1]], rows.at[0], gsem[0])
                    for j in range(_G):
                        if j + 1 < _G:
                            if j + 1 >= 4:
                                sd[j - 3].wait()
                            gd[j + 1] = pltpu.async_copy(
                                table.at[kig.at[2 * j + 3]],
                                rows.at[(j + 1) % 4], gsem[(j + 1) % 4])
                        gd[j].wait()
                        sd[j] = pltpu.async_copy(
                            rows.at[j % 4], zsh.at[kig.at[2 * j]],
                            ssem[j % 4], add=True)
                    for j in range(_G - 4, _G):
                        sd[j].wait()
                return 0
            lax.fori_loop(0, _TPG, tloop, 0)
            plsc.subcore_barrier()
            pltpu.sync_copy(zsh.at[pl.ds(sid * _ZROWS, _ZROWS)],
                            out.at[pl.ds(sid * _ZROWS, _ZROWS)])
            plsc.subcore_barrier()


def _sc_sparse(y4, yn4, ki1, ki2, zz):
    mesh = plsc.VectorSubcoreMesh(core_axis_name="c", subcore_axis_name="s")
    f = pl.kernel(
        _sc_body,
        out_type=[jax.ShapeDtypeStruct((4, _NPAD, _Q), jnp.float32),
                  jax.ShapeDtypeStruct((4, _NPAD, _Q), jnp.float32)],
        mesh=mesh,
        scratch_types=[
            pltpu.VMEM((2 * _G, _B), jnp.int32),
            pltpu.VMEM((4, _B, _Q), jnp.float32),
            pltpu.VMEM_SHARED((_NPAD, _Q), jnp.float32),
            pltpu.SemaphoreType.DMA,
            pltpu.SemaphoreType.DMA,
            pltpu.SemaphoreType.DMA,
            pltpu.SemaphoreType.DMA,
            pltpu.SemaphoreType.DMA,
            pltpu.SemaphoreType.DMA,
            pltpu.SemaphoreType.DMA,
            pltpu.SemaphoreType.DMA,
        ],
    )
    return f(y4, yn4, ki1, ki2, zz)


_BN = 1000


def _quarters(v):
    return jnp.stack([v[:, i * _Q:(i + 1) * _Q] for i in range(4)], axis=0)


def _init_body(op_ref, nf_ref, emb2_ref, wb_ref, vflat_ref, inv_ref,
               x_ref, y_ref, yn_ref):
    op = op_ref[...]
    onehot = (op == lax.broadcasted_iota(jnp.int32, (1, 128), 1)
              ).astype(jnp.float32)
    u = (jnp.dot(onehot, emb2_ref[...], preferred_element_type=jnp.float32)
         + jnp.dot(nf_ref[...], wb_ref[...], preferred_element_type=jnp.float32))
    x = jnp.concatenate([u] * 8, axis=1) + vflat_ref[...]
    x_ref[...] = x
    y = jnp.where(x >= 0, x, 0.2 * x)
    y_ref[...] = _quarters(y)
    yn_ref[...] = _quarters(y * inv_ref[...])


def _tc_init(op_ids2, node_feats, emb2, wb, vflat, inv2):
    n = node_feats.shape[0]
    grid = n // _BN
    outx = jax.ShapeDtypeStruct((n, _NFLAT), jnp.float32)
    out4 = jax.ShapeDtypeStruct((4, n, _Q), jnp.float32)
    blk4 = pl.BlockSpec((4, _BN, _Q), lambda i: (0, i, 0))
    return pl.pallas_call(
        _init_body,
        grid=(grid,),
        in_specs=[
            pl.BlockSpec((_BN, 1), lambda i: (i, 0)),
            pl.BlockSpec((_BN, 128), lambda i: (i, 0)),
            pl.BlockSpec((128, 64), lambda i: (0, 0)),
            pl.BlockSpec((128, 64), lambda i: (0, 0)),
            pl.BlockSpec((1, _NFLAT), lambda i: (0, 0)),
            pl.BlockSpec((_BN, 1), lambda i: (i, 0)),
        ],
        out_specs=[pl.BlockSpec((_BN, _NFLAT), lambda i: (i, 0)), blk4, blk4],
        out_shape=[outx, out4, out4],
    )(op_ids2, node_feats, emb2, wb, vflat, inv2)


def _upd_body(x_ref, s1_ref, s2_ref, y_ref, yn_ref, inv_ref,
              w1_ref, b1_ref, w2_ref, b2_ref, xn_ref, y2_ref, yn2_ref):
    act = lambda t: jnp.where(t >= 0, t, 0.2 * t)
    inv = inv_ref[...]
    cat = lambda a: jnp.concatenate([a[i] for i in range(4)], axis=-1)
    z = (cat(s1_ref[...]) + cat(s2_ref[...]) * inv
         + cat(y_ref[...]) + 2.0 * cat(yn_ref[...]))
    t = jnp.dot(act(jnp.dot(z, w1_ref[...], preferred_element_type=jnp.float32)
                    + b1_ref[...]),
                w2_ref[...], preferred_element_type=jnp.float32) + b2_ref[...]
    xn = x_ref[...] + t
    xn_ref[...] = xn
    y2 = act(xn)
    y2_ref[...] = _quarters(y2)
    yn2_ref[...] = _quarters(y2 * inv)


def _tc_update(x, s1, s2, y4, yn4, inv2, w1bd, b1t, w2bd, b2t):
    n = x.shape[0]
    grid = n // _BN
    outx = jax.ShapeDtypeStruct((n, _NFLAT), jnp.float32)
    out4 = jax.ShapeDtypeStruct((4, n, _Q), jnp.float32)
    blkx = pl.BlockSpec((_BN, _NFLAT), lambda i: (i, 0))
    blk4 = pl.BlockSpec((4, _BN, _Q), lambda i: (0, i, 0))
    return pl.pallas_call(
        _upd_body,
        grid=(grid,),
        in_specs=[
            blkx, blk4, blk4, blk4, blk4,
            pl.BlockSpec((_BN, 1), lambda i: (i, 0)),
            pl.BlockSpec((_NFLAT, _NFLAT), lambda i: (0, 0)),
            pl.BlockSpec((1, _NFLAT), lambda i: (0, 0)),
            pl.BlockSpec((_NFLAT, _NFLAT), lambda i: (0, 0)),
            pl.BlockSpec((1, _NFLAT), lambda i: (0, 0)),
        ],
        out_specs=[blkx, blk4, blk4],
        out_shape=[outx, out4, out4],
    )(x, s1, s2, y4, yn4, inv2, w1bd, b1t, w2bd, b2t)


def _pool_body(y_ref, out_ref):
    i = pl.program_id(0)

    @pl.when(i == 0)
    def _():
        out_ref[...] = jnp.zeros_like(out_ref)
    out_ref[...] += jnp.sum(y_ref[...], axis=1, keepdims=True)


def _tc_pool(y4):
    n = y4.shape[1]
    grid = n // _BN
    return pl.pallas_call(
        _pool_body,
        grid=(grid,),
        in_specs=[pl.BlockSpec((4, _BN, _Q), lambda i: (0, i, 0))],
        out_specs=pl.BlockSpec((4, 1, _Q), lambda i: (0, 0, 0)),
        out_shape=jax.ShapeDtypeStruct((4, 1, _Q), jnp.float32),
    )(y4)


def _post_body(pooled_ref, w1_ref, b1_ref, w2_ref, b2_ref, w3_ref, b3_ref,
               out_ref):
    act = lambda t: jnp.where(t >= 0, t, 0.2 * t)
    h = act(jnp.dot(pooled_ref[...], w1_ref[...],
                    preferred_element_type=jnp.float32) + b1_ref[...])
    h = act(jnp.dot(h, w2_ref[...], preferred_element_type=jnp.float32)
            + b2_ref[...])
    out_ref[...] = jnp.dot(h, w3_ref[...],
                           preferred_element_type=jnp.float32) + b3_ref[...]


def kernel(op_ids, node_feats, config_feats, edge_index, emb,
           gnn0_W, gnn0_b, gnn1_W, gnn1_b, gnn2_W, gnn2_b,
           shared_W1, shared_b1, shared_W2, shared_b2,
           post_W1, post_b1, post_W2, post_b2, post_W3, post_b3):
    n = node_feats.shape[0]
    c = config_feats.shape[0]
    ope = emb.shape[1]
    draw = node_feats.shape[1]

    w12 = gnn1_W @ gnn2_W
    wfold = gnn0_W @ w12
    bfold = gnn0_b @ w12 + gnn1_b @ gnn2_W + gnn2_b
    emb2 = emb @ wfold[:ope]
    wb = wfold[ope:ope + draw]
    vflat = (config_feats @ wfold[ope + draw:] + bfold).reshape(1, _NFLAT)

    src = edge_index[0].astype(jnp.int32)
    dst = edge_index[1].astype(jnp.int32)

    p2 = jnp.sort((dst << 14) | src)
    k2s = (p2 >> 14).astype(jnp.int32)
    i2s = (p2 & 16383).astype(jnp.int32)

    node_edges = jnp.searchsorted(
        k2s, jnp.arange(n + 1, dtype=jnp.int32)).astype(jnp.int32)
    deg = (node_edges[1:] - node_edges[:-1]).astype(jnp.float32) + 1.0
    inv2 = (1.0 / deg).reshape(n, 1)

    def interleave(k, i):
        kk = k.reshape(_NCH, 1, _B)
        ii = i.reshape(_NCH, 1, _B)
        return jnp.concatenate([kk, ii], axis=1).reshape(2 * _NCH, _B)

    ki1 = interleave(src, dst)
    ki2 = interleave(k2s, i2s)
    zz = jnp.zeros((_ZROWS, _Q), jnp.float32)

    eye = jnp.eye(c, dtype=jnp.float32)
    w1bd = jnp.kron(eye, shared_W1)
    w2bd = jnp.kron(eye, shared_W2)
    b1t = jnp.tile(shared_b1, c).reshape(1, _NFLAT)
    b2t = jnp.tile(shared_b2, c).reshape(1, _NFLAT)

    x, y4, yn4 = _tc_init(op_ids.reshape(n, 1).astype(jnp.int32),
                          node_feats, emb2, wb, vflat, inv2)

    for _ in range(3):
        s1, s2 = _sc_sparse(y4, yn4, ki1, ki2, zz)
        x, y4, yn4 = _tc_update(x, s1, s2, y4, yn4, inv2,
                                w1bd, b1t, w2bd, b2t)

    pooled = _tc_pool(y4).reshape(c, 64)

    out = pl.pallas_call(
        _post_body,
        out_shape=jax.ShapeDtypeStruct((c, 1), jnp.float32),
    )(pooled, post_W1, post_b1[None, :], post_W2, post_b2[None, :],
      post_W3, post_b3[None, :])
    return jnp.squeeze(out, -1)
